# baseline (speedup 1.0000x reference)
import functools

import jax
import jax.numpy as jnp
from jax import lax
from jax.experimental import pallas as pl
from jax.experimental.pallas import tpu as pltpu
from jax.experimental.pallas import tpu_sc as plsc

_C0 = 0.7978845608028654
_CA = 0.044715 * _C0
_STEP = 0.1
_CHUNK = 128
_L = 16
_ROWS = 8
_H = 512


def _tc_body(feat_ref, w1_ref, b1_ref, w2_ref, mask_ref, out_ref):
    f32 = jnp.float32
    w1m = w1_ref[0:768, :]
    wl = w1_ref[768:769, :]
    v = w2_ref[...] * wl
    hv = 0.5 * v
    wlv = wl * v
    c0 = b1_ref[...] + 0.5 * wl

    F = jnp.dot(feat_ref[...], w1m, preferred_element_type=f32)
    bf = jnp.bfloat16
    red1 = None
    red2 = None
    for k in range(768 // _CHUNK):
        sl = slice(k * _CHUNK, (k + 1) * _CHUNK)
        x = (F[:, sl] + c0[:, sl]).astype(bf)
        x2 = x * x
        t1 = bf(_CA) * x2 + bf(_C0)
        r = bf(3.0) * t1 - bf(2.0 * _C0)
        t = jnp.tanh(x * t1)
        s = bf(1.0) - t * t
        xsr = (x * s) * r
        a1 = jnp.sum((hv[:, sl].astype(bf) * (t + xsr)).astype(jnp.float32), axis=1, keepdims=True)
        g2 = s * ((bf(2.0) * r - bf(_C0)) - (x * t) * (r * r))
        a2 = jnp.sum((wlv[:, sl].astype(bf) * g2).astype(jnp.float32), axis=1, keepdims=True)
        red1 = a1 if red1 is None else red1 + a1
        red2 = a2 if red2 is None else red2 + a2

    sv = 0.5 * jnp.sum(v, axis=1, keepdims=True)
    d1 = red1 + sv
    m = mask_ref[...].astype(f32)
    p1 = (-_STEP * 0.25) * m * d1
    s2 = jax.nn.sigmoid(p1)
    d2 = d1 + (s2 - 0.5) * red2
    p2 = p1 - _STEP * (m * s2 * (1.0 - s2) * d2)
    rows = jnp.transpose(jnp.concatenate([p1, p2], axis=1), (1, 0))
    out_ref[...] = rows


def _lane_allreduce(x, op, buf):
    iota = lax.iota(jnp.int32, _L)
    for k in (1, 2, 4, 8):
        buf[...] = x
        y = plsc.load_gather(buf, [jnp.bitwise_xor(iota, k)])
        x = op(x, y)
    return x


def _sc_softmax(logits_hbm, mask_hbm, out_hbm, row_v, mask_v, e_v, red_v, sem):
    wid = lax.axis_index("s") * 2 + lax.axis_index("c")

    @pl.when(wid < _ROWS)
    def _():
        pltpu.sync_copy(logits_hbm.at[wid], row_v)
        pltpu.sync_copy(mask_hbm.at[wid % 4], mask_v)
        neg = jnp.full((_L,), -1e30, jnp.float32)
        macc = neg
        for k in range(_H // _L):
            x = row_v[pl.ds(k * _L, _L)]
            mk = mask_v[pl.ds(k * _L, _L)] > 0
            macc = jnp.maximum(macc, jnp.where(mk, x, neg))
        mx = _lane_allreduce(macc, jnp.maximum, red_v)
        sacc = jnp.zeros((_L,), jnp.float32)
        for k in range(_H // _L):
            x = row_v[pl.ds(k * _L, _L)]
            mk = mask_v[pl.ds(k * _L, _L)] > 0
            e = jnp.where(mk, jnp.exp(x - mx), 0.0)
            e_v[pl.ds(k * _L, _L)] = e
            sacc = sacc + e
        tot = _lane_allreduce(sacc, jnp.add, red_v)
        inv = 1.0 / jnp.maximum(tot, 1e-30)
        for k in range(_H // _L):
            e_v[pl.ds(k * _L, _L)] = e_v[pl.ds(k * _L, _L)] * inv
        pltpu.sync_copy(e_v, out_hbm.at[wid])


def kernel(features, attention_mask, training, W1, b1, W2, b2):
    B, H, D = features.shape
    N = B * H
    feat2d = features.reshape(N, D)
    b1row = b1.reshape(1, D)
    w2row = W2.reshape(1, D)
    maskcol = attention_mask.reshape(N, 1)

    logits = pl.pallas_call(
        _tc_body,
        grid=(B,),
        in_specs=[
            pl.BlockSpec((H, D), lambda i: (i, 0)),
            pl.BlockSpec((D + 1, D), lambda i: (0, 0)),
            pl.BlockSpec((1, D), lambda i: (0, 0)),
            pl.BlockSpec((1, D), lambda i: (0, 0)),
            pl.BlockSpec((H, 1), lambda i: (i, 0)),
        ],
        out_specs=pl.BlockSpec((2, H), lambda i: (0, i)),
        out_shape=jax.ShapeDtypeStruct((2, N), jnp.float32),
    )(feat2d, W1, b1row, w2row, maskcol)

    mesh = plsc.VectorSubcoreMesh(core_axis_name="c", subcore_axis_name="s")
    sc = pl.kernel(
        _sc_softmax,
        mesh=mesh,
        compiler_params=pltpu.CompilerParams(needs_layout_passes=False),
        out_type=jax.ShapeDtypeStruct((_ROWS, H), jnp.float32),
        scratch_types=[
            pltpu.VMEM((H,), jnp.float32),
            pltpu.VMEM((H,), jnp.int32),
            pltpu.VMEM((H,), jnp.float32),
            pltpu.VMEM((_L,), jnp.float32),
            pltpu.SemaphoreType.DMA,
        ],
    )
    probs = sc(logits.reshape(_ROWS, H), attention_mask)
    return probs.reshape(2, B, H, 1)


# speedup vs baseline: 2.2040x; 2.2040x over previous
import jax
import jax.numpy as jnp
from jax.experimental import pallas as pl

_C0 = 0.7978845608028654
_CA = 0.044715 * _C0
_STEP = 0.1
_CHUNK = 128


def _masked_softmax_col(p, m):
    lm = jnp.where(m, p, -1e30)
    mx = jnp.max(lm, axis=0, keepdims=True)
    e = jnp.where(m, jnp.exp(p - mx), 0.0)
    s = jnp.sum(e, axis=0, keepdims=True)
    return e / jnp.maximum(s, 1e-30)


def _body(feat_ref, w1_ref, b1_ref, w2_ref, mask_ref, out_ref):
    f32 = jnp.float32
    w1m = w1_ref[0:768, :]
    wl = w1_ref[768:769, :]
    v = w2_ref[...] * wl
    hv = 0.5 * v
    wlv = wl * v
    c0 = b1_ref[...] + 0.5 * wl

    F = jnp.dot(feat_ref[...], w1m, preferred_element_type=f32)

    bf = jnp.bfloat16
    c0b = _CA
    red1 = None
    red2 = None
    for k in range(768 // _CHUNK):
        sl = slice(k * _CHUNK, (k + 1) * _CHUNK)
        x = (F[:, sl] + c0[:, sl]).astype(bf)
        x2 = x * x
        t1 = bf(_CA) * x2 + bf(_C0)
        r = bf(3.0) * t1 - bf(2.0 * _C0)
        t = jnp.tanh(x * t1)
        s = bf(1.0) - t * t
        xsr = (x * s) * r
        a1 = jnp.sum((hv[:, sl].astype(bf) * (t + xsr)).astype(jnp.float32), axis=1, keepdims=True)
        g2 = s * ((bf(2.0) * r - bf(_C0)) - (x * t) * (r * r))
        a2 = jnp.sum((wlv[:, sl].astype(bf) * g2).astype(jnp.float32), axis=1, keepdims=True)
        red1 = a1 if red1 is None else red1 + a1
        red2 = a2 if red2 is None else red2 + a2

    sv = 0.5 * jnp.sum(v, axis=1, keepdims=True)
    d1 = red1 + sv
    m = mask_ref[...].astype(f32)
    p1 = (-_STEP * 0.25) * m * d1
    s2 = jax.nn.sigmoid(p1)
    delta = s2 - 0.5
    d2 = d1 + delta * red2
    p2 = p1 - _STEP * (m * s2 * (1.0 - s2) * d2)

    mb = m > 0.0
    q1 = _masked_softmax_col(p1, mb)
    q2 = _masked_softmax_col(p2, mb)
    rows = jnp.transpose(jnp.concatenate([q1, q2], axis=1), (1, 0))
    out_ref[...] = rows


def kernel(features, attention_mask, training, W1, b1, W2, b2):
    B, H, D = features.shape
    N = B * H
    feat2d = features.reshape(N, D)
    b1row = b1.reshape(1, D)
    w2row = W2.reshape(1, D)
    maskcol = attention_mask.reshape(N, 1)

    probs = pl.pallas_call(
        _body,
        grid=(B,),
        in_specs=[
            pl.BlockSpec((H, D), lambda i: (i, 0)),
            pl.BlockSpec((D + 1, D), lambda i: (0, 0)),
            pl.BlockSpec((1, D), lambda i: (0, 0)),
            pl.BlockSpec((1, D), lambda i: (0, 0)),
            pl.BlockSpec((H, 1), lambda i: (i, 0)),
        ],
        out_specs=pl.BlockSpec((2, H), lambda i: (0, i)),
        out_shape=jax.ShapeDtypeStruct((2, N), jnp.float32),
    )(feat2d, W1, b1row, w2row, maskcol)

    return probs.reshape(2, B, H, 1)
